# baseline (speedup 1.0000x reference)
import functools

import jax
import jax.numpy as jnp
from jax import lax
from jax.experimental import pallas as pl
from jax.experimental.pallas import tpu as pltpu
from jax.experimental.pallas import tpu_sc as plsc

_VOCAB = 1000000
_D = 64
_BATCH = 4096
_HIST = 200
_BB = 128
_TT = 2
_CH = _TT * _BB
_PW = 137


@functools.cache
def _build(nw: int):
    assert _BATCH // nw == _BB
    n_chunks = _HIST // _TT
    n_groups = n_chunks // 2
    b_per_w = _BB * _HIST
    mesh = plsc.VectorSubcoreMesh(core_axis_name="c", subcore_axis_name="s")

    @functools.partial(
        pl.kernel,
        mesh=mesh,
        out_type=jax.ShapeDtypeStruct((_HIST, _D, _BATCH), jnp.float32),
        compiler_params=pltpu.CompilerParams(
            use_tc_tiling_on_sc=False, needs_layout_passes=False
        ),
        scratch_types=[
            pltpu.VMEM((b_per_w,), jnp.int32),
            pltpu.VMEM((b_per_w,), jnp.int32),
            pltpu.VMEM((2, _CH, _D), jnp.float32),
            pltpu.VMEM((2, _TT, _D, _PW), jnp.float32),
            pltpu.SemaphoreType.DMA,
            pltpu.SemaphoreType.DMA,
            pltpu.SemaphoreType.DMA,
            pltpu.SemaphoreType.DMA,
        ],
    )
    def k(idx_hbm, table_hbm, out_hbm, stage_v, idxt_v, rows_v, patch_v,
          g0, g1, p0, p1):
        nc = 2
        wid = lax.axis_index("s") * nc + lax.axis_index("c")
        b0 = wid * _BB
        lanes = lax.iota(jnp.int32, 16)

        pltpu.sync_copy(idx_hbm.at[pl.ds(b0 * _HIST, b_per_w)], stage_v)

        def shuffle(t, _):
            for g in range(_BB // 16):
                src = (g * 16 + lanes) * _HIST + t
                vec = plsc.load_gather(stage_v, [src])
                idxt_v[pl.ds(t * _BB + g * 16, 16)] = vec
            return ()

        lax.fori_loop(0, _HIST, shuffle, (), unroll=False)

        gsems = (g0, g1)
        psems = (p0, p1)

        def gather(c, buf):
            pltpu.async_copy(
                table_hbm.at[idxt_v.at[pl.ds(c * _CH, _CH)]],
                rows_v.at[buf],
                gsems[buf],
            )

        def wait_gather(c, buf):
            pltpu.make_async_copy(
                table_hbm.at[idxt_v.at[pl.ds(c * _CH, _CH)]],
                rows_v.at[buf],
                gsems[buf],
            ).wait()

        def put(c, buf):
            pltpu.async_copy(
                patch_v.at[buf, :, :, pl.ds(0, _BB)],
                out_hbm.at[pl.ds(c * _TT, _TT), :, pl.ds(b0, _BB)],
                psems[buf],
            )

        def wait_put(c, buf):
            pltpu.make_async_copy(
                patch_v.at[buf, :, :, pl.ds(0, _BB)],
                out_hbm.at[pl.ds(c * _TT, _TT), :, pl.ds(b0, _BB)],
                psems[buf],
            ).wait()

        def transpose(buf):
            bufv = jnp.full((16,), buf, jnp.int32)

            @plsc.parallel_loop(0, _CH, unroll=8)
            def _(r):
                rv = jnp.full((16,), 0, jnp.int32) + r
                ttv = rv >> 7
                jv = rv & 127
                for fg in range(_D // 16):
                    vec = rows_v[buf, r, pl.ds(fg * 16, 16)]
                    plsc.store_scatter(
                        patch_v, [bufv, ttv, fg * 16 + lanes, jv], vec
                    )

        gather(0, 0)

        def body(grp, _):
            c = 2 * grp

            gather(c + 1, 1)
            wait_gather(c, 0)

            @pl.when(grp >= 1)
            def _():
                wait_put(c - 1, 1)

            transpose(0)
            put(c, 0)

            @pl.when(grp < n_groups - 1)
            def _():
                wait_put(c, 0)
                gather(c + 2, 0)

            wait_gather(c + 1, 1)
            transpose(1)
            put(c + 1, 1)
            return ()

        lax.fori_loop(0, n_groups, body, (), unroll=False)

        wait_put(n_chunks - 2, 0)
        wait_put(n_chunks - 1, 1)

    return k


def kernel(input, embeddings):
    idx = input.astype(jnp.int32).reshape(-1)
    out_t = _build(32)(idx, embeddings)
    return out_t.transpose(2, 0, 1)


# speedup vs baseline: 1.2739x; 1.2739x over previous
import functools

import jax
import jax.numpy as jnp
from jax import lax
from jax.experimental import pallas as pl
from jax.experimental.pallas import tpu as pltpu
from jax.experimental.pallas import tpu_sc as plsc

_VOCAB = 1000000
_D = 64
_BATCH = 4096
_HIST = 200
_BB = 128
_TT = 2
_CH = _TT * _BB
_PW = 137


@functools.cache
def _build(nw: int):
    assert _BATCH // nw == _BB
    n_chunks = _HIST // _TT
    n_groups = n_chunks // 2
    b_per_w = _BB * _HIST
    mesh = plsc.VectorSubcoreMesh(core_axis_name="c", subcore_axis_name="s")

    @functools.partial(
        pl.kernel,
        mesh=mesh,
        out_type=jax.ShapeDtypeStruct(
            (_HIST, _D // 8, _BATCH // _BB, 8, _BB), jnp.float32
        ),
        compiler_params=pltpu.CompilerParams(
            use_tc_tiling_on_sc=False, needs_layout_passes=False
        ),
        scratch_types=[
            pltpu.VMEM((b_per_w,), jnp.int32),
            pltpu.VMEM((b_per_w,), jnp.int32),
            pltpu.VMEM((2, _CH, _D), jnp.float32),
            pltpu.VMEM((2, _TT, _D // 8, 8, _PW), jnp.float32),
            pltpu.SemaphoreType.DMA,
            pltpu.SemaphoreType.DMA,
            pltpu.SemaphoreType.DMA,
            pltpu.SemaphoreType.DMA,
        ],
    )
    def k(idx_hbm, table_hbm, out_hbm, stage_v, idxt_v, rows_v, patch_v,
          g0, g1, p0, p1):
        nc = 2
        wid = lax.axis_index("s") * nc + lax.axis_index("c")
        b0 = wid * _BB
        lanes = lax.iota(jnp.int32, 16)

        pltpu.sync_copy(idx_hbm.at[pl.ds(b0 * _HIST, b_per_w)], stage_v)

        def shuffle(t, _):
            for g in range(_BB // 16):
                src = (g * 16 + lanes) * _HIST + t
                vec = plsc.load_gather(stage_v, [src])
                idxt_v[pl.ds(t * _BB + g * 16, 16)] = vec
            return ()

        lax.fori_loop(0, _HIST, shuffle, (), unroll=False)

        gsems = (g0, g1)
        psems = (p0, p1)

        def gather(c, buf):
            pltpu.async_copy(
                table_hbm.at[idxt_v.at[pl.ds(c * _CH, _CH)]],
                rows_v.at[buf],
                gsems[buf],
            )

        def wait_gather(c, buf):
            pltpu.make_async_copy(
                table_hbm.at[idxt_v.at[pl.ds(c * _CH, _CH)]],
                rows_v.at[buf],
                gsems[buf],
            ).wait()

        def put(c, buf):
            pltpu.async_copy(
                patch_v.at[buf, :, :, :, pl.ds(0, _BB)],
                out_hbm.at[pl.ds(c * _TT, _TT), :, wid, :, :],
                psems[buf],
            )

        def wait_put(c, buf):
            pltpu.make_async_copy(
                patch_v.at[buf, :, :, :, pl.ds(0, _BB)],
                out_hbm.at[pl.ds(c * _TT, _TT), :, wid, :, :],
                psems[buf],
            ).wait()

        def transpose(buf):
            bufv = jnp.full((16,), buf, jnp.int32)

            @plsc.parallel_loop(0, _CH, unroll=8)
            def _(r):
                rv = jnp.full((16,), 0, jnp.int32) + r
                ttv = rv >> 7
                jv = rv & 127
                for fg in range(_D // 16):
                    fvec = fg * 16 + lanes
                    vec = rows_v[buf, r, pl.ds(fg * 16, 16)]
                    plsc.store_scatter(
                        patch_v, [bufv, ttv, fvec >> 3, fvec & 7, jv], vec
                    )

        gather(0, 0)

        def body(grp, _):
            c = 2 * grp

            gather(c + 1, 1)
            wait_gather(c, 0)

            @pl.when(grp >= 1)
            def _():
                wait_put(c - 1, 1)

            transpose(0)
            put(c, 0)

            @pl.when(grp < n_groups - 1)
            def _():
                wait_put(c, 0)
                gather(c + 2, 0)

            wait_gather(c + 1, 1)
            transpose(1)
            put(c + 1, 1)
            return ()

        lax.fori_loop(0, n_groups, body, (), unroll=False)

        wait_put(n_chunks - 2, 0)
        wait_put(n_chunks - 1, 1)

    return k


def kernel(input, embeddings):
    idx = input.astype(jnp.int32).reshape(-1)
    out5 = _build(32)(idx, embeddings)
    return out5.transpose(2, 4, 0, 1, 3).reshape(_BATCH, _HIST, _D)
